# baseline (speedup 1.0000x reference)
import functools

import jax
import jax.numpy as jnp
from jax import lax
from jax.experimental import pallas as pl
from jax.experimental.pallas import tpu as pltpu
from jax.experimental.pallas import tpu_sc as plsc

D_MODEL = 64
SCALE = 8.0

NC = 2
NS = 16
L = 16
NW = NC * NS


@functools.cache
def _build(B0, B1):
    bw = B0 // NW
    assert bw % 128 == 0
    mesh = plsc.VectorSubcoreMesh(core_axis_name="c", subcore_axis_name="s")

    @functools.partial(
        pl.kernel,
        out_type=jax.ShapeDtypeStruct((B1, D_MODEL, B0), jnp.float32),
        mesh=mesh,
        compiler_params=pltpu.CompilerParams(
            use_tc_tiling_on_sc=False, needs_layout_passes=False),
        scratch_types=[
            pltpu.VMEM((B1, bw), jnp.int32),
            pltpu.VMEM((bw, D_MODEL), jnp.float32),
            pltpu.VMEM((bw, D_MODEL), jnp.float32),
            pltpu.VMEM((D_MODEL, bw), jnp.float32),
            pltpu.VMEM((D_MODEL, bw), jnp.float32),
            pltpu.SemaphoreType.DMA,
            pltpu.SemaphoreType.DMA,
            pltpu.SemaphoreType.DMA,
            pltpu.SemaphoreType.DMA,
        ],
    )
    def emb_kernel(xt_hbm, table_hbm, out_hbm, idx_v, rows0, rows1, t0, t1,
                   sem_g0, sem_g1, sem_s0, sem_s1):
        wid = lax.axis_index("s") * NC + lax.axis_index("c")
        col0 = pl.multiple_of(wid * bw, bw)
        rows = (rows0, rows1)
        tv = (t0, t1)
        sem_g = (sem_g0, sem_g1)
        sem_s = (sem_s0, sem_s1)

        pltpu.sync_copy(xt_hbm.at[:, pl.ds(col0, bw)], idx_v)

        def gather(b1, b):
            return pltpu.make_async_copy(
                table_hbm.at[idx_v.at[b1]], rows[b], sem_g[b])

        def store(b1, b):
            return pltpu.make_async_copy(
                tv[b], out_hbm.at[b1, :, pl.ds(col0, bw)], sem_s[b])

        lane = jax.lax.iota(jnp.int32, L)
        col_idx = [c * L + lane for c in range(D_MODEL // L)]

        gather(0, 0).start()

        @pl.loop(0, B1, step=2)
        def line(g):
            for tb in range(2):
                b1 = g + tb
                other = 1 - tb

                @pl.when(b1 + 1 < B1)
                def _fire_next_gather():
                    gather(b1 + 1, other).start()

                gather(b1, tb).wait()

                @pl.when(b1 >= 2)
                def _wait_prev_store():
                    store(b1 - 2, tb).wait()

                @plsc.parallel_loop(0, bw, unroll=4)
                def transpose_row(j):
                    jsplat = jnp.full((L,), j, jnp.int32)
                    for c in range(D_MODEL // L):
                        v = rows[tb][j, pl.ds(c * L, L)] * SCALE
                        plsc.store_scatter(tv[tb], [col_idx[c], jsplat], v)

                store(b1, tb).start()

        store(B1 - 2, 0).wait()
        store(B1 - 1, 1).wait()

    return emb_kernel


def kernel(x, table):
    B0, B1 = x.shape
    out_phys = _build(B0, B1)(x.T, table)
    return out_phys.transpose(2, 0, 1)


# speedup vs baseline: 1.0554x; 1.0554x over previous
import functools

import jax
import jax.numpy as jnp
from jax import lax
from jax.experimental import pallas as pl
from jax.experimental.pallas import tpu as pltpu
from jax.experimental.pallas import tpu_sc as plsc

D_MODEL = 64
SCALE = 8.0

NC = 2
NS = 16
L = 16
NW = NC * NS


@functools.cache
def _build(B0, B1):
    bw = B0 // NW
    assert bw == 128
    mesh = plsc.VectorSubcoreMesh(core_axis_name="c", subcore_axis_name="s")

    @functools.partial(
        pl.kernel,
        out_type=jax.ShapeDtypeStruct((B1, D_MODEL, B0), jnp.float32),
        mesh=mesh,
        compiler_params=pltpu.CompilerParams(
            use_tc_tiling_on_sc=False, needs_layout_passes=False),
        scratch_types=[
            pltpu.VMEM((B1, bw), jnp.int32),
            pltpu.VMEM((B1, bw), jnp.int32),
            pltpu.VMEM((bw, 2 * D_MODEL), jnp.float32),
            pltpu.VMEM((bw, 2 * D_MODEL), jnp.float32),
            pltpu.VMEM((D_MODEL, bw), jnp.float32),
            pltpu.VMEM((D_MODEL, bw), jnp.float32),
            pltpu.SemaphoreType.DMA,
            pltpu.SemaphoreType.DMA,
            pltpu.SemaphoreType.DMA,
            pltpu.SemaphoreType.DMA,
        ],
    )
    def emb_kernel(xt_hbm, table2_hbm, out_hbm, idx_v, par_v,
                   rows0, rows1, t0, t1, sem_g0, sem_g1, sem_s0, sem_s1):
        wid = lax.axis_index("s") * NC + lax.axis_index("c")
        col0 = pl.multiple_of(wid * bw, bw)
        rows = (rows0, rows1)
        tv = (t0, t1)
        sem_g = (sem_g0, sem_g1)
        sem_s = (sem_s0, sem_s1)

        pltpu.sync_copy(xt_hbm.at[:, pl.ds(col0, bw)], idx_v)

        @plsc.parallel_loop(0, B1)
        def split_idx(r):
            for c in range(bw // L):
                v = idx_v[r, pl.ds(c * L, L)]
                par_v[r, pl.ds(c * L, L)] = (v & 1) << 6
                idx_v[r, pl.ds(c * L, L)] = lax.shift_right_logical(v, 1)

        def gather(b1, b):
            return pltpu.make_async_copy(
                table2_hbm.at[idx_v.at[b1]], rows[b], sem_g[b])

        def store(b1, b):
            return pltpu.make_async_copy(
                tv[b], out_hbm.at[b1, :, pl.ds(col0, bw)], sem_s[b])

        lane = jax.lax.iota(jnp.int32, L)
        rot = [(lane + s) % L for s in range(L)]

        gather(0, 0).start()

        @pl.loop(0, B1, step=2)
        def line(g):
            for tb in range(2):
                b1 = g + tb
                other = 1 - tb

                @pl.when(b1 + 1 < B1)
                def _fire_next_gather():
                    gather(b1 + 1, other).start()

                gather(b1, tb).wait()

                @pl.when(b1 >= 2)
                def _wait_prev_store():
                    store(b1 - 2, tb).wait()

                @plsc.parallel_loop(0, bw // L)
                def transpose_block(jj):
                    j0 = pl.multiple_of(jj * L, L)
                    row_idx = j0 + lane
                    par = par_v[b1, pl.ds(j0, L)]
                    for c in range(D_MODEL // L):
                        for s in range(L):
                            base = c * L + rot[s]
                            v = plsc.load_gather(
                                rows[tb], [row_idx, base + par])
                            plsc.store_scatter(
                                tv[tb], [base, row_idx], v * SCALE)

                store(b1, tb).start()

        store(B1 - 2, 0).wait()
        store(B1 - 1, 1).wait()

    return emb_kernel


def kernel(x, table):
    B0, B1 = x.shape
    V, D = table.shape
    table2 = table.reshape(V // 2, 2 * D)
    out_phys = _build(B0, B1)(x.T, table2)
    return out_phys.transpose(2, 0, 1)


# speedup vs baseline: 1.2202x; 1.1562x over previous
import functools

import jax
import jax.numpy as jnp
from jax import lax
from jax.experimental import pallas as pl
from jax.experimental.pallas import tpu as pltpu
from jax.experimental.pallas import tpu_sc as plsc

D_MODEL = 64
SCALE = 8.0

NC = 2
NS = 16
L = 16
NW = NC * NS


@functools.cache
def _build(B0, B1):
    bw = B0 // NW
    assert bw == 128 and B1 == 200
    mesh = plsc.VectorSubcoreMesh(core_axis_name="c", subcore_axis_name="s")

    @functools.partial(
        pl.kernel,
        out_type=jax.ShapeDtypeStruct((B0 * B1, D_MODEL), jnp.float32),
        mesh=mesh,
        compiler_params=pltpu.CompilerParams(
            use_tc_tiling_on_sc=False, needs_layout_passes=False),
        scratch_types=[
            pltpu.VMEM((B1, bw), jnp.int32),
            pltpu.VMEM((bw, B1), jnp.int32),
            pltpu.VMEM((B1, D_MODEL), jnp.float32),
            pltpu.VMEM((B1, D_MODEL), jnp.float32),
            pltpu.SemaphoreType.DMA,
            pltpu.SemaphoreType.DMA,
            pltpu.SemaphoreType.DMA,
            pltpu.SemaphoreType.DMA,
        ],
    )
    def emb_kernel(xt_hbm, table_hbm, out_hbm, xst_v, idx_v,
                   rows0, rows1, sem_g0, sem_g1, sem_s0, sem_s1):
        wid = lax.axis_index("s") * NC + lax.axis_index("c")
        col0 = pl.multiple_of(wid * bw, bw)
        rows = (rows0, rows1)
        sem_g = (sem_g0, sem_g1)
        sem_s = (sem_s0, sem_s1)

        pltpu.sync_copy(xt_hbm.at[:, pl.ds(col0, bw)], xst_v)

        lane = jax.lax.iota(jnp.int32, L)

        @plsc.parallel_loop(0, B1)
        def transpose_idx(b1):
            bsplat = jnp.full((L,), b1, jnp.int32)
            for k in range(bw // L):
                v = xst_v[b1, pl.ds(k * L, L)]
                plsc.store_scatter(idx_v, [k * L + lane, bsplat], v)

        def gather(j, b):
            return (
                pltpu.make_async_copy(
                    table_hbm.at[idx_v.at[j, pl.ds(0, 128)]],
                    rows[b].at[pl.ds(0, 128)], sem_g[b]),
                pltpu.make_async_copy(
                    table_hbm.at[idx_v.at[j, pl.ds(128, B1 - 128)]],
                    rows[b].at[pl.ds(128, B1 - 128)], sem_g[b]),
            )

        def store(j, b):
            return pltpu.make_async_copy(
                rows[b], out_hbm.at[pl.ds((col0 + j) * B1, B1)], sem_s[b])

        for c in gather(0, 0):
            c.start()

        @pl.loop(0, bw, step=2)
        def line(g):
            for tb in range(2):
                j = g + tb
                other = 1 - tb

                @pl.when(j + 1 < bw)
                def _fire_next_gather():
                    for c in gather(j + 1, other):
                        c.start()

                for c in gather(j, tb):
                    c.wait()

                @pl.when(j >= 2)
                def _wait_prev_store():
                    store(j - 2, tb).wait()

                @plsc.parallel_loop(0, B1, unroll=4)
                def scale_row(r):
                    for c in range(D_MODEL // L):
                        v = rows[tb][r, pl.ds(c * L, L)]
                        rows[tb][r, pl.ds(c * L, L)] = v * SCALE

                store(j, tb).start()

        store(bw - 2, 0).wait()
        store(bw - 1, 1).wait()

    return emb_kernel


def kernel(x, table):
    B0, B1 = x.shape
    out = _build(B0, B1)(x.T, table)
    return out.reshape(B0, B1, D_MODEL)
